# baseline (speedup 1.0000x reference)
import functools
import jax
import jax.numpy as jnp
from jax import lax
from jax.experimental import pallas as pl
from jax.experimental.pallas import tpu as pltpu
from jax.experimental.pallas import tpu_sc as plsc

SEQ = 2048
E = 768
NH = 12
HS = 64
FF = 3072
NEXP = 8
VOCAB = 8192
EPS = 1e-5

TOK_BLK = 256
TILE = 512
NT_MAX = 16
PTOT = NT_MAX * TILE
FFC = 1536
NFFC = FF // FFC
VBLK = 1024


def _ln(x, s, b):
    mu = jnp.mean(x, axis=-1, keepdims=True)
    var = jnp.mean((x - mu) ** 2, axis=-1, keepdims=True)
    return (x - mu) * jax.lax.rsqrt(var + EPS) * s + b


def _t1_body(emb_ref, pos_ref, s_ref, b_ref, w_ref, wb_ref, x1_ref, qkv_ref):
    x = _ln(emb_ref[...] + pos_ref[...], s_ref[...], b_ref[...])
    x1_ref[...] = x
    qkv_ref[...] = jnp.dot(x, w_ref[...],
                           preferred_element_type=jnp.float32) + wb_ref[...]


def _t1(emb, pos, ln1_s, ln1_b, wqkv, bqkv):
    n = SEQ // TOK_BLK
    return pl.pallas_call(
        _t1_body,
        grid=(n,),
        in_specs=[
            pl.BlockSpec((TOK_BLK, E), lambda i: (i, 0)),
            pl.BlockSpec((TOK_BLK, E), lambda i: (i, 0)),
            pl.BlockSpec((1, E), lambda i: (0, 0)),
            pl.BlockSpec((1, E), lambda i: (0, 0)),
            pl.BlockSpec((E, 3 * E), lambda i: (0, 0)),
            pl.BlockSpec((1, 3 * E), lambda i: (0, 0)),
        ],
        out_specs=[
            pl.BlockSpec((TOK_BLK, E), lambda i: (i, 0)),
            pl.BlockSpec((TOK_BLK, 3 * E), lambda i: (i, 0)),
        ],
        out_shape=[
            jax.ShapeDtypeStruct((SEQ, E), jnp.float32),
            jax.ShapeDtypeStruct((SEQ, 3 * E), jnp.float32),
        ],
    )(emb, pos, ln1_s.reshape(1, E), ln1_b.reshape(1, E), wqkv, bqkv)


def _t2_body(q_ref, k_ref, v_ref, o_ref):
    i = pl.program_id(1)
    q = q_ref[0]
    s = lax.dot_general(q, k_ref[0], (((1,), (1,)), ((), ())),
                        preferred_element_type=jnp.float32)
    s = s * (1.0 / (E ** 0.5))
    rows = i * TOK_BLK + lax.broadcasted_iota(jnp.int32, (TOK_BLK, SEQ), 0)
    cols = lax.broadcasted_iota(jnp.int32, (TOK_BLK, SEQ), 1)
    s = jnp.where(rows >= cols, s, -1e30)
    m = jnp.max(s, axis=1, keepdims=True)
    p = jnp.exp(s - m)
    p = p / jnp.sum(p, axis=1, keepdims=True)
    o_ref[0] = jnp.dot(p, v_ref[0], preferred_element_type=jnp.float32)


def _t2(qkv3):
    n = SEQ // TOK_BLK
    return pl.pallas_call(
        _t2_body,
        grid=(NH, n),
        in_specs=[
            pl.BlockSpec((1, TOK_BLK, HS), lambda h, i: (h, i, 0)),
            pl.BlockSpec((1, SEQ, HS), lambda h, i: (NH + h, 0, 0)),
            pl.BlockSpec((1, SEQ, HS), lambda h, i: (2 * NH + h, 0, 0)),
        ],
        out_specs=pl.BlockSpec((1, TOK_BLK, HS), lambda h, i: (h, i, 0)),
        out_shape=jax.ShapeDtypeStruct((NH, SEQ, HS), jnp.float32),
    )(qkv3, qkv3, qkv3)


def _t3_body(o_ref, x1_ref, wp_ref, bp_ref, s2_ref, b2_ref, rw1_ref, rb1_ref,
             rw2_ref, rb2_ref, x3_ref, sc_ref):
    x2 = jnp.dot(o_ref[...], wp_ref[...],
                 preferred_element_type=jnp.float32) + bp_ref[...] + x1_ref[...]
    x3 = _ln(x2, s2_ref[...], b2_ref[...])
    x3_ref[...] = x3
    h = jnp.maximum(jnp.dot(x3, rw1_ref[...],
                            preferred_element_type=jnp.float32) + rb1_ref[...], 0.0)
    sc_ref[...] = jnp.dot(h, rw2_ref[...],
                          preferred_element_type=jnp.float32) + rb2_ref[...]


def _t3(o, x1, Wp, bp, ln2_s, ln2_b, rW1, rb1, rW2, rb2):
    n = SEQ // TOK_BLK
    return pl.pallas_call(
        _t3_body,
        grid=(n,),
        in_specs=[
            pl.BlockSpec((TOK_BLK, E), lambda i: (i, 0)),
            pl.BlockSpec((TOK_BLK, E), lambda i: (i, 0)),
            pl.BlockSpec((E, E), lambda i: (0, 0)),
            pl.BlockSpec((1, E), lambda i: (0, 0)),
            pl.BlockSpec((1, E), lambda i: (0, 0)),
            pl.BlockSpec((1, E), lambda i: (0, 0)),
            pl.BlockSpec((E, FF), lambda i: (0, 0)),
            pl.BlockSpec((1, FF), lambda i: (0, 0)),
            pl.BlockSpec((FF, NEXP), lambda i: (0, 0)),
            pl.BlockSpec((1, NEXP), lambda i: (0, 0)),
        ],
        out_specs=[
            pl.BlockSpec((TOK_BLK, E), lambda i: (i, 0)),
            pl.BlockSpec((TOK_BLK, NEXP), lambda i: (i, 0)),
        ],
        out_shape=[
            jax.ShapeDtypeStruct((SEQ, E), jnp.float32),
            jax.ShapeDtypeStruct((SEQ, NEXP), jnp.float32),
        ],
    )(o, x1, Wp, bp.reshape(1, E), ln2_s.reshape(1, E), ln2_b.reshape(1, E),
      rW1, rb1.reshape(1, FF), rW2, rb2.reshape(1, NEXP))


def _t4_body(sc_ref, gates_ref, dests_ref, tile_e_ref, nact_ref):
    s = sc_ref[...]
    io8 = lax.broadcasted_iota(jnp.int32, (SEQ, NEXP), 1)
    m1 = jnp.max(s, axis=1, keepdims=True)
    i1 = jnp.min(jnp.where(s == m1, io8, NEXP), axis=1, keepdims=True)
    s2 = jnp.where(io8 == i1, -jnp.float32(1e30), s)
    m2 = jnp.max(s2, axis=1, keepdims=True)
    i2 = jnp.min(jnp.where(s2 == m2, io8, NEXP), axis=1, keepdims=True)
    t = jnp.exp(m2 - m1)
    g1 = 1.0 / (1.0 + t)
    g2 = t / (1.0 + t)
    gates_ref[...] = jnp.concatenate([g1, g2], axis=1)

    oh0 = (io8 == i1).astype(jnp.float32)
    oh1 = (io8 == i2).astype(jnp.float32)
    oh = oh0 + oh1
    r = lax.broadcasted_iota(jnp.int32, (SEQ, SEQ), 0)
    c = lax.broadcasted_iota(jnp.int32, (SEQ, SEQ), 1)
    L = (r > c).astype(jnp.float32)
    excl = jnp.dot(L, oh, preferred_element_type=jnp.float32)
    pos0 = jnp.sum(excl * oh0, axis=1, keepdims=True)
    pos1 = jnp.sum(excl * oh1, axis=1, keepdims=True)

    cnt = jnp.sum(oh, axis=0, keepdims=True)
    cnti = cnt.astype(jnp.int32)
    nt_e = (cnti + (TILE - 1)) // TILE
    padded = (nt_e * TILE).astype(jnp.float32)
    re = lax.broadcasted_iota(jnp.int32, (NEXP, NEXP), 0)
    ce = lax.broadcasted_iota(jnp.int32, (NEXP, NEXP), 1)
    U = (re < ce).astype(jnp.float32)
    offs = jnp.dot(padded, U, preferred_element_type=jnp.float32)
    d0 = jnp.sum(offs * oh0, axis=1, keepdims=True) + pos0
    d1 = jnp.sum(offs * oh1, axis=1, keepdims=True) + pos1
    dests_ref[...] = jnp.concatenate([d0, d1], axis=1).astype(jnp.int32)

    tile_off = (offs.astype(jnp.int32)) // TILE
    ntot = jnp.sum(nt_e)
    nact_ref[0, 0] = ntot
    jt = lax.broadcasted_iota(jnp.int32, (NT_MAX, NEXP), 0)
    je = lax.broadcasted_iota(jnp.int32, (NT_MAX, NEXP), 1)
    act = jnp.logical_and(jt >= tile_off, jt < tile_off + nt_e)
    te = jnp.sum(jnp.where(act, je, 0), axis=1)
    e_last = jnp.max(jnp.where(nt_e > 0, lax.broadcasted_iota(
        jnp.int32, (1, NEXP), 1), -1))
    jrow = lax.broadcasted_iota(jnp.int32, (NT_MAX,), 0)
    tile_e_ref[...] = jnp.where(jrow < ntot, te, e_last)[None, :]


def _t4(score):
    return pl.pallas_call(
        _t4_body,
        out_specs=[
            pl.BlockSpec((SEQ, 2), lambda: (0, 0)),
            pl.BlockSpec((SEQ, 2), lambda: (0, 0)),
            pl.BlockSpec((1, NT_MAX), lambda: (0, 0)),
            pl.BlockSpec(memory_space=pltpu.SMEM),
        ],
        out_shape=[
            jax.ShapeDtypeStruct((SEQ, 2), jnp.float32),
            jax.ShapeDtypeStruct((SEQ, 2), jnp.int32),
            jax.ShapeDtypeStruct((1, NT_MAX), jnp.int32),
            jax.ShapeDtypeStruct((1, 1), jnp.int32),
        ],
    )(score)


def _t5_body(tile_e_ref, nact_ref, xs_ref, w1_ref, b1_ref, w2_ref, b2_ref,
             y_ref):
    i = pl.program_id(0)
    j = pl.program_id(1)

    @pl.when(i < nact_ref[0])
    def _():
        h = jnp.maximum(
            jnp.dot(xs_ref[...], w1_ref[0],
                    preferred_element_type=jnp.float32) + b1_ref[0], 0.0)
        part = jnp.dot(h, w2_ref[0], preferred_element_type=jnp.float32)

        @pl.when(j == 0)
        def _():
            y_ref[...] = part + b2_ref[0]

        @pl.when(j > 0)
        def _():
            y_ref[...] += part


def _t5(tile_e, nact, xs, eW1, eb1, eW2, eb2):
    grid_spec = pltpu.PrefetchScalarGridSpec(
        num_scalar_prefetch=2,
        grid=(NT_MAX, NFFC),
        in_specs=[
            pl.BlockSpec((TILE, E), lambda i, j, te, na: (i, 0)),
            pl.BlockSpec((1, E, FFC), lambda i, j, te, na: (te[i], 0, j)),
            pl.BlockSpec((1, 1, FFC), lambda i, j, te, na: (te[i], 0, j)),
            pl.BlockSpec((1, FFC, E), lambda i, j, te, na: (te[i], j, 0)),
            pl.BlockSpec((1, 1, E), lambda i, j, te, na: (te[i], 0, 0)),
        ],
        out_specs=pl.BlockSpec((TILE, E), lambda i, j, te, na: (i, 0)),
    )
    return pl.pallas_call(
        _t5_body,
        grid_spec=grid_spec,
        out_shape=jax.ShapeDtypeStruct((PTOT, E), jnp.float32),
    )(tile_e, nact, xs, eW1, eb1.reshape(NEXP, 1, FF), eW2,
      eb2.reshape(NEXP, 1, E))


def _t6_body(x3_ref, y0_ref, y1_ref, g_ref, s_ref, b_ref, w_ref, wb_ref,
             out_ref):
    g = g_ref[...]
    x4 = (g[:, 0:1] * y0_ref[...] + g[:, 1:2] * y1_ref[...] + x3_ref[...])
    xf = _ln(x4, s_ref[...], b_ref[...])
    out_ref[...] = jnp.dot(xf, w_ref[...],
                           preferred_element_type=jnp.float32) + wb_ref[...]


def _t6(x3, y0, y1, gates, lnf_s, lnf_b, Wout, bout):
    n = SEQ // TOK_BLK
    nv = VOCAB // VBLK
    return pl.pallas_call(
        _t6_body,
        grid=(nv, n),
        in_specs=[
            pl.BlockSpec((TOK_BLK, E), lambda v, i: (i, 0)),
            pl.BlockSpec((TOK_BLK, E), lambda v, i: (i, 0)),
            pl.BlockSpec((TOK_BLK, E), lambda v, i: (i, 0)),
            pl.BlockSpec((TOK_BLK, 2), lambda v, i: (i, 0)),
            pl.BlockSpec((1, E), lambda v, i: (0, 0)),
            pl.BlockSpec((1, E), lambda v, i: (0, 0)),
            pl.BlockSpec((E, VBLK), lambda v, i: (0, v)),
            pl.BlockSpec((1, VBLK), lambda v, i: (0, v)),
        ],
        out_specs=pl.BlockSpec((TOK_BLK, VBLK), lambda v, i: (i, v)),
        out_shape=jax.ShapeDtypeStruct((SEQ, VOCAB), jnp.float32),
    )(x3, y0, y1, gates, lnf_s.reshape(1, E), lnf_b.reshape(1, E), Wout,
      bout.reshape(1, VOCAB))


_MESH = dict(core_axis_name="c", subcore_axis_name="s")
_NW = 32
_LANES = 16


def _s1_emb(tok_emb, ids):
    bpw = SEQ // _NW

    @functools.partial(
        pl.kernel,
        mesh=plsc.VectorSubcoreMesh(**_MESH),
        out_type=jax.ShapeDtypeStruct((SEQ, E), jnp.float32),
        scratch_types=[
            pltpu.VMEM((bpw,), jnp.int32),
            pltpu.VMEM((bpw, E), jnp.float32),
            pltpu.SemaphoreType.DMA,
        ],
    )
    def k(tab_hbm, idx_hbm, out_hbm, idx_v, rows_v, sem):
        wid = lax.axis_index("s") * 2 + lax.axis_index("c")
        base = wid * bpw
        pltpu.sync_copy(idx_hbm.at[pl.ds(base, bpw)], idx_v)
        pltpu.async_copy(tab_hbm.at[idx_v], rows_v, sem).wait()
        pltpu.sync_copy(rows_v, out_hbm.at[pl.ds(base, bpw)])

    return k(tok_emb, ids)


def _s2_scatter(dests):
    nch = SEQ // _LANES

    @functools.partial(
        pl.kernel,
        mesh=plsc.VectorSubcoreMesh(**_MESH),
        out_type=jax.ShapeDtypeStruct((PTOT,), jnp.int32),
        scratch_types=[
            pltpu.VMEM((PTOT,), jnp.int32),
            pltpu.VMEM((SEQ,), jnp.int32),
            pltpu.VMEM((SEQ,), jnp.int32),
        ],
        compiler_params=pltpu.CompilerParams(needs_layout_passes=False),
    )
    def k(d0_hbm, d1_hbm, out_hbm, st_v, d0_v, d1_v):
        wid = lax.axis_index("s") * 2 + lax.axis_index("c")

        @pl.when(wid == 0)
        def _():
            zero = jnp.zeros((_LANES,), jnp.int32)
            for q in range(PTOT // _LANES):
                st_v[pl.ds(q * _LANES, _LANES)] = zero
            pltpu.sync_copy(d0_hbm, d0_v)
            pltpu.sync_copy(d1_hbm, d1_v)
            io = lax.iota(jnp.int32, _LANES)
            for q in range(nch):
                toks = io + (q * _LANES)
                plsc.store_scatter(st_v, [d0_v[pl.ds(q * _LANES, _LANES)]],
                                   toks)
                plsc.store_scatter(st_v, [d1_v[pl.ds(q * _LANES, _LANES)]],
                                   toks)
            pltpu.sync_copy(st_v, out_hbm)

    d0 = dests[:, 0].reshape(SEQ)
    d1 = dests[:, 1].reshape(SEQ)
    return k(d0, d1)


def _s3_gather_rows(src, idx, nrows, ncols):
    bpw = nrows // _NW
    chunk = 64
    nloop = bpw // chunk

    @functools.partial(
        pl.kernel,
        mesh=plsc.VectorSubcoreMesh(**_MESH),
        out_type=jax.ShapeDtypeStruct((nrows, ncols), jnp.float32),
        scratch_types=[
            pltpu.VMEM((bpw,), jnp.int32),
            pltpu.VMEM((chunk, ncols), jnp.float32),
            pltpu.SemaphoreType.DMA,
        ],
    )
    def k(src_hbm, idx_hbm, out_hbm, idx_v, rows_v, sem):
        wid = lax.axis_index("s") * 2 + lax.axis_index("c")
        base = wid * bpw
        pltpu.sync_copy(idx_hbm.at[pl.ds(base, bpw)], idx_v)
        for q in range(nloop):
            pltpu.async_copy(src_hbm.at[idx_v.at[pl.ds(q * chunk, chunk)]],
                             rows_v, sem).wait()
            pltpu.sync_copy(rows_v, out_hbm.at[pl.ds(base + q * chunk, chunk)])

    return k(src, idx)


def kernel(inputs, tok_emb, pos_emb, ln1_s, ln1_b, ln2_s, ln2_b, Wq, bq, Wk,
           bk, Wv, bv, Wp, bp, rW1, rb1, rW2, rb2, eW1, eb1, eW2, eb2, lnf_s,
           lnf_b, Wout, bout):
    ids = inputs.reshape(SEQ).astype(jnp.int32)

    wq = jnp.transpose(Wq, (1, 0, 2)).reshape(E, NH * HS)
    wk = jnp.transpose(Wk, (1, 0, 2)).reshape(E, NH * HS)
    wv = jnp.transpose(Wv, (1, 0, 2)).reshape(E, NH * HS)
    wqkv = jnp.concatenate([wq, wk, wv], axis=1)
    bqkv = jnp.concatenate([bq.reshape(1, NH * HS), bk.reshape(1, NH * HS),
                            bv.reshape(1, NH * HS)], axis=1)

    emb = _s1_emb(tok_emb, ids)
    x1, qkv = _t1(emb, pos_emb, ln1_s, ln1_b, wqkv, bqkv)
    qkv3 = qkv.reshape(SEQ, 3 * NH, HS).transpose(1, 0, 2)
    o3 = _t2(qkv3)
    o = o3.transpose(1, 0, 2).reshape(SEQ, E)
    x3, score = _t3(o, x1, Wp, bp, ln2_s, ln2_b, rW1, rb1, rW2, rb2)
    gates, dests, tile_e, nact = _t4(score)

    sorted_tok = _s2_scatter(dests)
    xs = _s3_gather_rows(x3, sorted_tok, PTOT, E)
    y = _t5(tile_e.reshape(NT_MAX), nact.reshape(1), xs, eW1, eb1, eW2, eb2)
    y0 = _s3_gather_rows(y, dests[:, 0].reshape(SEQ), SEQ, E)
    y1 = _s3_gather_rows(y, dests[:, 1].reshape(SEQ), SEQ, E)

    logits = _t6(x3, y0, y1, gates, lnf_s, lnf_b, Wout, bout)
    return logits.reshape(1, SEQ, VOCAB)
